# baseline (device time: 51080 ns/iter reference)
import jax
import jax.numpy as jnp
from jax import lax
from jax.experimental import pallas as pl
from jax.experimental.pallas import tpu as pltpu

B, S, H, Dh, Dr, D = 2, 256, 16, 64, 32, 1024
DC = 64
SCALE = (Dh + Dr) ** -0.5
N_CHUNK = 4
CS = S // N_CHUNK
MESH = pl.DeviceIdType.MESH


def kernel(x, Wdkv, Wuk, Wuv, Wq, Wqr, Wkr, Wo):
    def body(x_ref, wdkv_ref, wuk_ref, wuv_ref, wq_ref, wqr_ref, wkr_ref,
             wo_ref, out_ref, wdkv_recv, wuk_recv, wuv_recv, o_ref,
             w_send_sems, w_recv_sems, o_send_sems, o_recv_sems):
        my_x = lax.axis_index("x")
        my_y = lax.axis_index("y")
        my_z = lax.axis_index("z")
        xpartner = (1 - my_x, my_y, my_z)
        ypartner = (my_x, 1 - my_y, my_z)
        my_b = my_y

        barrier_sem = pltpu.get_barrier_semaphore()
        pl.semaphore_signal(barrier_sem, inc=1, device_id=xpartner,
                            device_id_type=MESH)
        pl.semaphore_signal(barrier_sem, inc=1, device_id=ypartner,
                            device_id_type=MESH)
        pl.semaphore_wait(barrier_sem, 2)

        w_rdmas = []
        for i, (src, dst) in enumerate([(wdkv_ref, wdkv_recv),
                                        (wuk_ref, wuk_recv),
                                        (wuv_ref, wuv_recv)]):
            r = pltpu.make_async_remote_copy(
                src_ref=src, dst_ref=dst,
                send_sem=w_send_sems.at[i], recv_sem=w_recv_sems.at[i],
                device_id=xpartner, device_id_type=MESH,
            )
            r.start()
            w_rdmas.append(r)

        xb = x_ref[pl.ds(my_b, 1)].reshape(S, D)
        q_all = jnp.dot(xb, wq_ref[...], preferred_element_type=jnp.float32)
        qr_all = jnp.dot(xb, wqr_ref[...], preferred_element_type=jnp.float32)
        kr_b = jnp.dot(xb, wkr_ref[...], preferred_element_type=jnp.float32)

        for r in w_rdmas:
            r.wait()

        c1 = jnp.dot(xb, wdkv_ref[...], preferred_element_type=jnp.float32)
        c2 = jnp.dot(xb, wdkv_recv[...], preferred_element_type=jnp.float32)
        k_all = (jnp.dot(c1, wuk_ref[...], preferred_element_type=jnp.float32)
                 + jnp.dot(c2, wuk_recv[...],
                           preferred_element_type=jnp.float32))
        v_all = (jnp.dot(c1, wuv_ref[...], preferred_element_type=jnp.float32)
                 + jnp.dot(c2, wuv_recv[...],
                           preferred_element_type=jnp.float32))

        for h in range(H):
            c0, c1_ = h * Dh, (h + 1) * Dh
            q = q_all[:, c0:c1_]
            k = k_all[:, c0:c1_]
            v = v_all[:, c0:c1_]
            qr = qr_all[:, h * Dr:(h + 1) * Dr]
            s = (lax.dot_general(q, k, (((1,), (1,)), ((), ())),
                                 preferred_element_type=jnp.float32)
                 + lax.dot_general(qr, kr_b, (((1,), (1,)), ((), ())),
                                   preferred_element_type=jnp.float32)
                 ) * SCALE
            m = jnp.max(s, axis=-1, keepdims=True)
            e = jnp.exp(s - m)
            p = e / jnp.sum(e, axis=-1, keepdims=True)
            o_ref[:, c0:c1_] = jnp.dot(p, v, preferred_element_type=jnp.float32)

        o_rdmas = []
        for j in range(N_CHUNK):
            chunk = jnp.dot(o_ref[j * CS:(j + 1) * CS, :], wo_ref[...],
                            preferred_element_type=jnp.float32)
            out_ref[pl.ds(my_b, 1), pl.ds(j * CS, CS), :] = chunk[None]
            r = pltpu.make_async_remote_copy(
                src_ref=out_ref.at[pl.ds(my_b, 1), pl.ds(j * CS, CS), :],
                dst_ref=out_ref.at[pl.ds(my_b, 1), pl.ds(j * CS, CS), :],
                send_sem=o_send_sems.at[j], recv_sem=o_recv_sems.at[j],
                device_id=ypartner, device_id_type=MESH,
            )
            r.start()
            o_rdmas.append(r)

        for r in o_rdmas:
            r.wait()

    return pl.pallas_call(
        body,
        out_shape=jax.ShapeDtypeStruct((B, S, D), jnp.float32),
        in_specs=[pl.BlockSpec(memory_space=pltpu.VMEM)] * 8,
        out_specs=pl.BlockSpec(memory_space=pltpu.VMEM),
        scratch_shapes=[
            pltpu.VMEM((D, DC), jnp.float32),
            pltpu.VMEM((DC, D), jnp.float32),
            pltpu.VMEM((DC, D), jnp.float32),
            pltpu.VMEM((S, D), jnp.float32),
            pltpu.SemaphoreType.DMA((3,)),
            pltpu.SemaphoreType.DMA((3,)),
            pltpu.SemaphoreType.DMA((N_CHUNK,)),
            pltpu.SemaphoreType.DMA((N_CHUNK,)),
        ],
        compiler_params=pltpu.CompilerParams(collective_id=0),
    )(x, Wdkv, Wuk, Wuv, Wq, Wqr, Wkr, Wo)


# device time: 50896 ns/iter; 1.0036x vs baseline; 1.0036x over previous
import jax
import jax.numpy as jnp
from jax import lax
from jax.experimental import pallas as pl
from jax.experimental.pallas import tpu as pltpu

B, S, H, Dh, Dr, D = 2, 256, 16, 64, 32, 1024
DC = 64
SCALE = (Dh + Dr) ** -0.5
N_CHUNK = 4
CS = S // N_CHUNK
MESH = pl.DeviceIdType.MESH


def kernel(x, Wdkv, Wuk, Wuv, Wq, Wqr, Wkr, Wo):
    def body(x_ref, wdkv_ref, wuk_ref, wuv_ref, wq_ref, wqr_ref, wkr_ref,
             wo_ref, out_ref, wdkv_recv, wuk_recv, wuv_recv, o_ref,
             w_send_sems, w_recv_sems, o_send_sems, o_recv_sems):
        my_x = lax.axis_index("x")
        my_y = lax.axis_index("y")
        my_z = lax.axis_index("z")
        xpartner = (1 - my_x, my_y, my_z)
        ypartner = (my_x, 1 - my_y, my_z)
        my_b = my_y

        barrier_sem = pltpu.get_barrier_semaphore()
        pl.semaphore_signal(barrier_sem, inc=1, device_id=xpartner,
                            device_id_type=MESH)
        pl.semaphore_signal(barrier_sem, inc=1, device_id=ypartner,
                            device_id_type=MESH)
        pl.semaphore_wait(barrier_sem, 2)

        w_rdmas = []
        for i, (src, dst) in enumerate([(wdkv_ref, wdkv_recv),
                                        (wuk_ref, wuk_recv),
                                        (wuv_ref, wuv_recv)]):
            r = pltpu.make_async_remote_copy(
                src_ref=src, dst_ref=dst,
                send_sem=w_send_sems.at[i], recv_sem=w_recv_sems.at[i],
                device_id=xpartner, device_id_type=MESH,
            )
            r.start()
            w_rdmas.append(r)

        def run_batch(b):
            xb = x_ref[b]
            q_all = jnp.dot(xb, wq_ref[...],
                            preferred_element_type=jnp.float32)
            qr_all = jnp.dot(xb, wqr_ref[...],
                             preferred_element_type=jnp.float32)
            kr_b = jnp.dot(xb, wkr_ref[...],
                           preferred_element_type=jnp.float32)

            for r in w_rdmas:
                r.wait()

            c1 = jnp.dot(xb, wdkv_ref[...],
                         preferred_element_type=jnp.float32)
            c2 = jnp.dot(xb, wdkv_recv[...],
                         preferred_element_type=jnp.float32)
            k_all = (jnp.dot(c1, wuk_ref[...],
                             preferred_element_type=jnp.float32)
                     + jnp.dot(c2, wuk_recv[...],
                               preferred_element_type=jnp.float32))
            v_all = (jnp.dot(c1, wuv_ref[...],
                             preferred_element_type=jnp.float32)
                     + jnp.dot(c2, wuv_recv[...],
                               preferred_element_type=jnp.float32))

            for h in range(H):
                c0, c1_ = h * Dh, (h + 1) * Dh
                q = q_all[:, c0:c1_]
                k = k_all[:, c0:c1_]
                v = v_all[:, c0:c1_]
                qr = qr_all[:, h * Dr:(h + 1) * Dr]
                s = (lax.dot_general(q, k, (((1,), (1,)), ((), ())),
                                     preferred_element_type=jnp.float32)
                     + lax.dot_general(qr, kr_b, (((1,), (1,)), ((), ())),
                                       preferred_element_type=jnp.float32)
                     ) * SCALE
                m = jnp.max(s, axis=-1, keepdims=True)
                e = jnp.exp(s - m)
                p = e / jnp.sum(e, axis=-1, keepdims=True)
                o_ref[:, c0:c1_] = jnp.dot(
                    p, v, preferred_element_type=jnp.float32)

            o_rdmas = []
            for j in range(N_CHUNK):
                chunk = jnp.dot(o_ref[j * CS:(j + 1) * CS, :], wo_ref[...],
                                preferred_element_type=jnp.float32)
                out_ref[b, j * CS:(j + 1) * CS, :] = chunk
                r = pltpu.make_async_remote_copy(
                    src_ref=out_ref.at[b, pl.ds(j * CS, CS), :],
                    dst_ref=out_ref.at[b, pl.ds(j * CS, CS), :],
                    send_sem=o_send_sems.at[j], recv_sem=o_recv_sems.at[j],
                    device_id=ypartner, device_id_type=MESH,
                )
                r.start()
                o_rdmas.append(r)

            for r in o_rdmas:
                r.wait()

        @pl.when(my_b == 0)
        def _():
            run_batch(0)

        @pl.when(my_b == 1)
        def _():
            run_batch(1)

    return pl.pallas_call(
        body,
        out_shape=jax.ShapeDtypeStruct((B, S, D), jnp.float32),
        in_specs=[pl.BlockSpec(memory_space=pltpu.VMEM)] * 8,
        out_specs=pl.BlockSpec(memory_space=pltpu.VMEM),
        scratch_shapes=[
            pltpu.VMEM((D, DC), jnp.float32),
            pltpu.VMEM((DC, D), jnp.float32),
            pltpu.VMEM((DC, D), jnp.float32),
            pltpu.VMEM((S, D), jnp.float32),
            pltpu.SemaphoreType.DMA((3,)),
            pltpu.SemaphoreType.DMA((3,)),
            pltpu.SemaphoreType.DMA((N_CHUNK,)),
            pltpu.SemaphoreType.DMA((N_CHUNK,)),
        ],
        compiler_params=pltpu.CompilerParams(collective_id=0),
    )(x, Wdkv, Wuk, Wuv, Wq, Wqr, Wkr, Wo)


# device time: 40227 ns/iter; 1.2698x vs baseline; 1.2652x over previous
import jax
import jax.numpy as jnp
from jax import lax
from jax.experimental import pallas as pl
from jax.experimental.pallas import tpu as pltpu

B, S, H, Dh, Dr, D = 2, 256, 16, 64, 32, 1024
T = B * S
DC = 64
SCALE = (Dh + Dr) ** -0.5
MESH = pl.DeviceIdType.MESH
BF = jnp.bfloat16
F32 = jnp.float32


def _dot(a, b):
    return jnp.dot(a, b, preferred_element_type=F32)


def kernel(x, Wdkv, Wuk, Wuv, Wq, Wqr, Wkr, Wo):
    def body(x_ref, wdkv_ref, wuk_ref, wuv_ref, wq_ref, wqr_ref, wkr_ref,
             wo_ref, out_ref, c_ref, c_recv, wuk16_ref, wuk_recv,
             wuv16_ref, wuv_recv, o_ref, send_sems, recv_sems):
        my_x = lax.axis_index("x")
        my_y = lax.axis_index("y")
        my_z = lax.axis_index("z")
        partner = (1 - my_x, my_y, my_z)

        xf = x_ref[...].reshape(T, D)
        x16 = xf.astype(BF)
        wdkv16 = wdkv_ref[...].astype(BF)
        wuk16_ref[...] = wuk_ref[...].astype(BF)
        wuv16_ref[...] = wuv_ref[...].astype(BF)

        c_ref[...] = _dot(x16, wdkv16).astype(BF)

        barrier_sem = pltpu.get_barrier_semaphore()
        pl.semaphore_signal(barrier_sem, inc=1, device_id=partner,
                            device_id_type=MESH)
        pl.semaphore_wait(barrier_sem, 1)

        rdmas = []
        for i, (src, dst) in enumerate([(c_ref, c_recv),
                                        (wuk16_ref, wuk_recv),
                                        (wuv16_ref, wuv_recv)]):
            r = pltpu.make_async_remote_copy(
                src_ref=src, dst_ref=dst,
                send_sem=send_sems.at[i], recv_sem=recv_sems.at[i],
                device_id=partner, device_id_type=MESH,
            )
            r.start()
            rdmas.append(r)

        q_all = _dot(x16, wq_ref[...].astype(BF))
        qr_all = _dot(x16, wqr_ref[...].astype(BF))
        kr_all = _dot(x16, wkr_ref[...].astype(BF))

        for r in rdmas:
            r.wait()
        c_loc = c_ref[...]
        c_rem = c_recv[...]
        k_all = _dot(c_loc, wuk16_ref[...]) + _dot(c_rem, wuk_recv[...])
        v_all = _dot(c_loc, wuv16_ref[...]) + _dot(c_rem, wuv_recv[...])

        for b in range(B):
            kr_b = kr_all[b * S:(b + 1) * S, :].astype(BF)
            for h in range(H):
                r0, r1 = b * S, (b + 1) * S
                c0, c1 = h * Dh, (h + 1) * Dh
                q = q_all[r0:r1, c0:c1].astype(BF)
                k = k_all[r0:r1, c0:c1].astype(BF)
                v = v_all[r0:r1, c0:c1].astype(BF)
                qr = qr_all[r0:r1, h * Dr:(h + 1) * Dr].astype(BF)
                s = (lax.dot_general(q, k, (((1,), (1,)), ((), ())),
                                     preferred_element_type=F32)
                     + lax.dot_general(qr, kr_b, (((1,), (1,)), ((), ())),
                                       preferred_element_type=F32)
                     ) * SCALE
                m = jnp.max(s, axis=-1, keepdims=True)
                e = jnp.exp(s - m)
                p = e / jnp.sum(e, axis=-1, keepdims=True)
                o_ref[r0:r1, c0:c1] = _dot(p.astype(BF), v)

        out = _dot(o_ref[...].astype(BF), wo_ref[...].astype(BF))
        out_ref[...] = out.reshape(B, S, D)

    return pl.pallas_call(
        body,
        out_shape=jax.ShapeDtypeStruct((B, S, D), F32),
        in_specs=[pl.BlockSpec(memory_space=pltpu.VMEM)] * 8,
        out_specs=pl.BlockSpec(memory_space=pltpu.VMEM),
        scratch_shapes=[
            pltpu.VMEM((T, DC), BF),
            pltpu.VMEM((T, DC), BF),
            pltpu.VMEM((DC, D), BF),
            pltpu.VMEM((DC, D), BF),
            pltpu.VMEM((DC, D), BF),
            pltpu.VMEM((DC, D), BF),
            pltpu.VMEM((T, D), F32),
            pltpu.SemaphoreType.DMA((3,)),
            pltpu.SemaphoreType.DMA((3,)),
        ],
        compiler_params=pltpu.CompilerParams(collective_id=0),
    )(x, Wdkv, Wuk, Wuv, Wq, Wqr, Wkr, Wo)


# device time: 33739 ns/iter; 1.5140x vs baseline; 1.1923x over previous
import jax
import jax.numpy as jnp
from jax import lax
from jax.experimental import pallas as pl
from jax.experimental.pallas import tpu as pltpu

B, S, H, Dh, Dr, D = 2, 256, 16, 64, 32, 1024
T = B * S
DC = 64
SCALE = (Dh + Dr) ** -0.5
MESH = pl.DeviceIdType.MESH
BF = jnp.bfloat16
F32 = jnp.float32


def _dot(a, b):
    return jnp.dot(a, b, preferred_element_type=F32)


def kernel(x, Wdkv, Wuk, Wuv, Wq, Wqr, Wkr, Wo):
    def body(x_ref, wdkv_ref, wuk_ref, wuv_ref, wkr_ref,
             wq_hbm, wqr_hbm, wo_hbm, out_ref,
             wq_v, wqr_v, wo_v, c_ref, c_recv, wuk16_ref, wuk_recv,
             wuv16_ref, wuv_recv, o_ref, copy_sems, send_sems, recv_sems):
        my_x = lax.axis_index("x")
        my_y = lax.axis_index("y")
        my_z = lax.axis_index("z")
        partner = (1 - my_x, my_y, my_z)

        copies = []
        for i, (src, dst) in enumerate([(wq_hbm, wq_v), (wqr_hbm, wqr_v),
                                        (wo_hbm, wo_v)]):
            cp = pltpu.make_async_copy(src, dst, copy_sems.at[i])
            cp.start()
            copies.append(cp)

        barrier_sem = pltpu.get_barrier_semaphore()
        pl.semaphore_signal(barrier_sem, inc=1, device_id=partner,
                            device_id_type=MESH)

        xf = x_ref[...].reshape(T, D)
        x16 = xf.astype(BF)
        wuk16_ref[...] = wuk_ref[...].astype(BF)
        wuv16_ref[...] = wuv_ref[...].astype(BF)
        c_ref[...] = _dot(x16, wdkv_ref[...].astype(BF)).astype(BF)

        pl.semaphore_wait(barrier_sem, 1)
        rdmas = []
        for i, (src, dst) in enumerate([(c_ref, c_recv),
                                        (wuk16_ref, wuk_recv),
                                        (wuv16_ref, wuv_recv)]):
            r = pltpu.make_async_remote_copy(
                src_ref=src, dst_ref=dst,
                send_sem=send_sems.at[i], recv_sem=recv_sems.at[i],
                device_id=partner, device_id_type=MESH,
            )
            r.start()
            rdmas.append(r)

        xq = xf * SCALE
        copies[0].wait()
        q_all = _dot(xq, wq_v[...])
        copies[1].wait()
        qr_all = _dot(xq, wqr_v[...])
        kr_all = _dot(xf, wkr_ref[...])

        for r in rdmas:
            r.wait()
        c_loc = c_ref[...]
        c_rem = c_recv[...]
        k_all = _dot(c_loc, wuk16_ref[...]) + _dot(c_rem, wuk_recv[...])
        v_all = _dot(c_loc, wuv16_ref[...]) + _dot(c_rem, wuv_recv[...])

        for b in range(B):
            kr_b = kr_all[b * S:(b + 1) * S, :]
            for h in range(H):
                r0, r1 = b * S, (b + 1) * S
                c0, c1 = h * Dh, (h + 1) * Dh
                q = q_all[r0:r1, c0:c1]
                k = k_all[r0:r1, c0:c1]
                v = v_all[r0:r1, c0:c1]
                qr = qr_all[r0:r1, h * Dr:(h + 1) * Dr]
                s = (lax.dot_general(q, k, (((1,), (1,)), ((), ())),
                                     preferred_element_type=F32)
                     + lax.dot_general(qr, kr_b, (((1,), (1,)), ((), ())),
                                       preferred_element_type=F32))
                e = jnp.exp(s)
                denom = jnp.sum(e, axis=-1, keepdims=True)
                o_ref[r0:r1, c0:c1] = _dot(e, v) / denom

        copies[2].wait()
        out = _dot(o_ref[...], wo_v[...])
        out_ref[...] = out.reshape(B, S, D)

    return pl.pallas_call(
        body,
        out_shape=jax.ShapeDtypeStruct((B, S, D), F32),
        in_specs=[
            pl.BlockSpec(memory_space=pltpu.VMEM),
            pl.BlockSpec(memory_space=pltpu.VMEM),
            pl.BlockSpec(memory_space=pltpu.VMEM),
            pl.BlockSpec(memory_space=pltpu.VMEM),
            pl.BlockSpec(memory_space=pltpu.VMEM),
            pl.BlockSpec(memory_space=pltpu.HBM),
            pl.BlockSpec(memory_space=pltpu.HBM),
            pl.BlockSpec(memory_space=pltpu.HBM),
        ],
        out_specs=pl.BlockSpec(memory_space=pltpu.VMEM),
        scratch_shapes=[
            pltpu.VMEM((D, D), F32),
            pltpu.VMEM((D, H * Dr), F32),
            pltpu.VMEM((D, D), F32),
            pltpu.VMEM((T, DC), BF),
            pltpu.VMEM((T, DC), BF),
            pltpu.VMEM((DC, D), BF),
            pltpu.VMEM((DC, D), BF),
            pltpu.VMEM((DC, D), BF),
            pltpu.VMEM((DC, D), BF),
            pltpu.VMEM((T, D), F32),
            pltpu.SemaphoreType.DMA((3,)),
            pltpu.SemaphoreType.DMA((3,)),
            pltpu.SemaphoreType.DMA((3,)),
        ],
        compiler_params=pltpu.CompilerParams(collective_id=0),
    )(x, Wdkv, Wuk, Wuv, Wkr, Wq, Wqr, Wo)


# device time: 32668 ns/iter; 1.5636x vs baseline; 1.0328x over previous
import jax
import jax.numpy as jnp
from jax import lax
from jax.experimental import pallas as pl
from jax.experimental.pallas import tpu as pltpu

B, S, H, Dh, Dr, D = 2, 256, 16, 64, 32, 1024
T = B * S
DC = 64
HH = H // 2
HD = HH * Dh
SCALE = (Dh + Dr) ** -0.5
MESH = pl.DeviceIdType.MESH
BF = jnp.bfloat16
F32 = jnp.float32


def _dot(a, b):
    return jnp.dot(a, b, preferred_element_type=F32)


def kernel(x, Wdkv, Wuk, Wuv, Wq, Wqr, Wkr, Wo):
    def body(x_ref, wdkv_ref, wuk_ref, wuv_ref, wkr_ref,
             wq_hbm, wqr_hbm, wo_hbm, out_ref,
             wq_v, wqr_v, wo_v, c_ref, c_recv, wuk16_ref, wuk_recv,
             wuv16_ref, wuv_recv, o16_ref, copy_sems, send_sems,
             recv_sems, o_send_sems, o_recv_sems):
        my_x = lax.axis_index("x")
        my_y = lax.axis_index("y")
        my_z = lax.axis_index("z")
        xpartner = (1 - my_x, my_y, my_z)
        ypartner = (my_x, 1 - my_y, my_z)

        wo_cp = pltpu.make_async_copy(wo_hbm, wo_v, copy_sems.at[2])
        wo_cp.start()

        barrier_sem = pltpu.get_barrier_semaphore()
        pl.semaphore_signal(barrier_sem, inc=1, device_id=xpartner,
                            device_id_type=MESH)
        pl.semaphore_signal(barrier_sem, inc=1, device_id=ypartner,
                            device_id_type=MESH)

        xf = x_ref[...].reshape(T, D)
        x16 = xf.astype(BF)
        c_ref[...] = _dot(x16, wdkv_ref[...].astype(BF)).astype(BF)

        def run_half(hy):
            hc0, hc1 = hy * HD, (hy + 1) * HD
            rc0, rc1 = hy * HH * Dr, (hy + 1) * HH * Dr

            wq_cp = pltpu.make_async_copy(
                wq_hbm.at[:, hc0:hc1], wq_v, copy_sems.at[0])
            wq_cp.start()
            wqr_cp = pltpu.make_async_copy(
                wqr_hbm.at[:, rc0:rc1], wqr_v, copy_sems.at[1])
            wqr_cp.start()

            wuk16_ref[...] = wuk_ref[:, hc0:hc1].astype(BF)
            wuv16_ref[...] = wuv_ref[:, hc0:hc1].astype(BF)

            pl.semaphore_wait(barrier_sem, 2)
            rdmas = []
            for i, (src, dst) in enumerate([(c_ref, c_recv),
                                            (wuk16_ref, wuk_recv),
                                            (wuv16_ref, wuv_recv)]):
                r = pltpu.make_async_remote_copy(
                    src_ref=src, dst_ref=dst,
                    send_sem=send_sems.at[i], recv_sem=recv_sems.at[i],
                    device_id=xpartner, device_id_type=MESH,
                )
                r.start()
                rdmas.append(r)

            xq = xf * SCALE
            wq_cp.wait()
            q_all = _dot(xq, wq_v[...])
            wqr_cp.wait()
            qr_all = _dot(xq, wqr_v[...])
            kr_all = _dot(xf, wkr_ref[...])

            for r in rdmas:
                r.wait()
            c_loc = c_ref[...]
            c_rem = c_recv[...]
            k_all = _dot(c_loc, wuk16_ref[...]) + _dot(c_rem, wuk_recv[...])
            v_all = _dot(c_loc, wuv16_ref[...]) + _dot(c_rem, wuv_recv[...])

            o_rdmas = []
            for b in range(B):
                kr_b = kr_all[b * S:(b + 1) * S, :]
                for i in range(HH):
                    j = b * HH + i
                    r0, r1 = b * S, (b + 1) * S
                    lc0, lc1 = i * Dh, (i + 1) * Dh
                    gc0 = hc0 + lc0
                    q = q_all[r0:r1, lc0:lc1]
                    k = k_all[r0:r1, lc0:lc1]
                    v = v_all[r0:r1, lc0:lc1]
                    qr = qr_all[r0:r1, i * Dr:(i + 1) * Dr]
                    s = (lax.dot_general(q, k, (((1,), (1,)), ((), ())),
                                         preferred_element_type=F32)
                         + lax.dot_general(qr, kr_b,
                                           (((1,), (1,)), ((), ())),
                                           preferred_element_type=F32))
                    e = jnp.exp(s)
                    denom = jnp.sum(e, axis=-1, keepdims=True)
                    o16_ref[r0:r1, gc0:gc0 + Dh] = (_dot(e, v)
                                                    / denom).astype(BF)
                    if i % 2 == 1:
                        jj = b * (HH // 2) + i // 2
                        pc0 = gc0 - Dh
                        r = pltpu.make_async_remote_copy(
                            src_ref=o16_ref.at[r0:r1, pc0:pc0 + 2 * Dh],
                            dst_ref=o16_ref.at[r0:r1, pc0:pc0 + 2 * Dh],
                            send_sem=o_send_sems.at[jj],
                            recv_sem=o_recv_sems.at[jj],
                            device_id=ypartner, device_id_type=MESH,
                        )
                        r.start()
                        o_rdmas.append(r)

            for r in o_rdmas:
                r.wait()

        @pl.when(my_y == 0)
        def _():
            run_half(0)

        @pl.when(my_y == 1)
        def _():
            run_half(1)

        wo_cp.wait()
        out = _dot(o16_ref[...], wo_v[...])
        out_ref[...] = out.reshape(B, S, D)

    return pl.pallas_call(
        body,
        out_shape=jax.ShapeDtypeStruct((B, S, D), F32),
        in_specs=[
            pl.BlockSpec(memory_space=pltpu.VMEM),
            pl.BlockSpec(memory_space=pltpu.VMEM),
            pl.BlockSpec(memory_space=pltpu.VMEM),
            pl.BlockSpec(memory_space=pltpu.VMEM),
            pl.BlockSpec(memory_space=pltpu.VMEM),
            pl.BlockSpec(memory_space=pltpu.HBM),
            pl.BlockSpec(memory_space=pltpu.HBM),
            pl.BlockSpec(memory_space=pltpu.HBM),
        ],
        out_specs=pl.BlockSpec(memory_space=pltpu.VMEM),
        scratch_shapes=[
            pltpu.VMEM((D, HD), F32),
            pltpu.VMEM((D, HH * Dr), F32),
            pltpu.VMEM((D, D), F32),
            pltpu.VMEM((T, DC), BF),
            pltpu.VMEM((T, DC), BF),
            pltpu.VMEM((DC, HD), BF),
            pltpu.VMEM((DC, HD), BF),
            pltpu.VMEM((DC, HD), BF),
            pltpu.VMEM((DC, HD), BF),
            pltpu.VMEM((T, D), BF),
            pltpu.SemaphoreType.DMA((3,)),
            pltpu.SemaphoreType.DMA((3,)),
            pltpu.SemaphoreType.DMA((3,)),
            pltpu.SemaphoreType.DMA((HH,)),
            pltpu.SemaphoreType.DMA((HH,)),
        ],
        compiler_params=pltpu.CompilerParams(collective_id=0),
    )(x, Wdkv, Wuk, Wuv, Wkr, Wq, Wqr, Wo)


# device time: 32207 ns/iter; 1.5860x vs baseline; 1.0143x over previous
import jax
import jax.numpy as jnp
from jax import lax
from jax.experimental import pallas as pl
from jax.experimental.pallas import tpu as pltpu

B, S, H, Dh, Dr, D = 2, 256, 16, 64, 32, 1024
T = B * S
DC = 64
HH = H // 2
HD = HH * Dh
SCALE = (Dh + Dr) ** -0.5
MESH = pl.DeviceIdType.MESH
BF = jnp.bfloat16
F32 = jnp.float32


def _dot(a, b):
    return jnp.dot(a, b, preferred_element_type=F32)


def kernel(x, Wdkv, Wuk, Wuv, Wq, Wqr, Wkr, Wo):
    def body(x_ref, wdkv_ref, wuk_ref, wuv_ref, wkr_ref,
             wq_hbm, wqr_hbm, wo_hbm, out_ref,
             wq_v, wqr_v, wo_v, c_ref, c_recv, wuk16_ref, wuk_recv,
             wuv16_ref, wuv_recv, o16_ref, out_v, copy_sems, send_sems,
             recv_sems, o_send_sems, o_recv_sems, out_sems):
        my_x = lax.axis_index("x")
        my_y = lax.axis_index("y")
        my_z = lax.axis_index("z")
        xpartner = (1 - my_x, my_y, my_z)
        ypartner = (my_x, 1 - my_y, my_z)

        wo_cp = pltpu.make_async_copy(wo_hbm, wo_v, copy_sems.at[2])
        wo_cp.start()

        barrier_sem = pltpu.get_barrier_semaphore()
        pl.semaphore_signal(barrier_sem, inc=1, device_id=xpartner,
                            device_id_type=MESH)
        pl.semaphore_signal(barrier_sem, inc=1, device_id=ypartner,
                            device_id_type=MESH)

        xf = x_ref[...].reshape(T, D)
        x16 = xf.astype(BF)
        c_ref[...] = _dot(x16, wdkv_ref[...].astype(BF)).astype(BF)

        def run_half(hy):
            hc0, hc1 = hy * HD, (hy + 1) * HD
            rc0, rc1 = hy * HH * Dr, (hy + 1) * HH * Dr

            wq_cp = pltpu.make_async_copy(
                wq_hbm.at[:, hc0:hc1], wq_v, copy_sems.at[0])
            wq_cp.start()
            wqr_cp = pltpu.make_async_copy(
                wqr_hbm.at[:, rc0:rc1], wqr_v, copy_sems.at[1])
            wqr_cp.start()

            wuk16_ref[...] = wuk_ref[:, hc0:hc1].astype(BF)
            wuv16_ref[...] = wuv_ref[:, hc0:hc1].astype(BF)

            pl.semaphore_wait(barrier_sem, 2)
            rdmas = []
            for i, (src, dst) in enumerate([(c_ref, c_recv),
                                            (wuk16_ref, wuk_recv),
                                            (wuv16_ref, wuv_recv)]):
                r = pltpu.make_async_remote_copy(
                    src_ref=src, dst_ref=dst,
                    send_sem=send_sems.at[i], recv_sem=recv_sems.at[i],
                    device_id=xpartner, device_id_type=MESH,
                )
                r.start()
                rdmas.append(r)

            xq = xf * SCALE
            wq_cp.wait()
            q_all = _dot(xq, wq_v[...])
            wqr_cp.wait()
            qr_all = _dot(xq, wqr_v[...])
            kr_all = _dot(xf, wkr_ref[...])

            for r in rdmas:
                r.wait()
            c_loc = c_ref[...]
            c_rem = c_recv[...]
            k_all = _dot(c_loc, wuk16_ref[...]) + _dot(c_rem, wuk_recv[...])
            v_all = _dot(c_loc, wuv16_ref[...]) + _dot(c_rem, wuv_recv[...])

            o_rdmas = []
            for b in range(B):
                kr_b = kr_all[b * S:(b + 1) * S, :].astype(BF)
                for i in range(HH):
                    r0, r1 = b * S, (b + 1) * S
                    lc0, lc1 = i * Dh, (i + 1) * Dh
                    gc0 = hc0 + lc0
                    q = q_all[r0:r1, lc0:lc1].astype(BF)
                    k = k_all[r0:r1, lc0:lc1].astype(BF)
                    v = v_all[r0:r1, lc0:lc1].astype(BF)
                    qr = qr_all[r0:r1, i * Dr:(i + 1) * Dr].astype(BF)
                    s = (lax.dot_general(q, k, (((1,), (1,)), ((), ())),
                                         preferred_element_type=F32)
                         + lax.dot_general(qr, kr_b,
                                           (((1,), (1,)), ((), ())),
                                           preferred_element_type=F32))
                    e = jnp.exp(s.astype(BF))
                    denom = jnp.sum(e, axis=-1, keepdims=True, dtype=F32)
                    o16_ref[r0:r1, gc0:gc0 + Dh] = (_dot(e, v)
                                                    / denom).astype(BF)
                    if i % 2 == 1:
                        jj = b * (HH // 2) + i // 2
                        pc0 = gc0 - Dh
                        r = pltpu.make_async_remote_copy(
                            src_ref=o16_ref.at[r0:r1, pc0:pc0 + 2 * Dh],
                            dst_ref=o16_ref.at[r0:r1, pc0:pc0 + 2 * Dh],
                            send_sem=o_send_sems.at[jj],
                            recv_sem=o_recv_sems.at[jj],
                            device_id=ypartner, device_id_type=MESH,
                        )
                        r.start()
                        o_rdmas.append(r)

            for r in o_rdmas:
                r.wait()

        @pl.when(my_y == 0)
        def _():
            run_half(0)

        @pl.when(my_y == 1)
        def _():
            run_half(1)

        wo_cp.wait()
        out_cps = []
        for b in range(B):
            out_v[b] = _dot(o16_ref[b * S:(b + 1) * S, :], wo_v[...])
            cp = pltpu.make_async_copy(out_v.at[b], out_ref.at[b],
                                       out_sems.at[b])
            cp.start()
            out_cps.append(cp)
        for cp in out_cps:
            cp.wait()

    return pl.pallas_call(
        body,
        out_shape=jax.ShapeDtypeStruct((B, S, D), F32),
        in_specs=[
            pl.BlockSpec(memory_space=pltpu.VMEM),
            pl.BlockSpec(memory_space=pltpu.VMEM),
            pl.BlockSpec(memory_space=pltpu.VMEM),
            pl.BlockSpec(memory_space=pltpu.VMEM),
            pl.BlockSpec(memory_space=pltpu.VMEM),
            pl.BlockSpec(memory_space=pltpu.HBM),
            pl.BlockSpec(memory_space=pltpu.HBM),
            pl.BlockSpec(memory_space=pltpu.HBM),
        ],
        out_specs=pl.BlockSpec(memory_space=pltpu.HBM),
        scratch_shapes=[
            pltpu.VMEM((D, HD), F32),
            pltpu.VMEM((D, HH * Dr), F32),
            pltpu.VMEM((D, D), F32),
            pltpu.VMEM((T, DC), BF),
            pltpu.VMEM((T, DC), BF),
            pltpu.VMEM((DC, HD), BF),
            pltpu.VMEM((DC, HD), BF),
            pltpu.VMEM((DC, HD), BF),
            pltpu.VMEM((DC, HD), BF),
            pltpu.VMEM((T, D), BF),
            pltpu.VMEM((B, S, D), F32),
            pltpu.SemaphoreType.DMA((3,)),
            pltpu.SemaphoreType.DMA((3,)),
            pltpu.SemaphoreType.DMA((3,)),
            pltpu.SemaphoreType.DMA((HH,)),
            pltpu.SemaphoreType.DMA((HH,)),
            pltpu.SemaphoreType.DMA((B,)),
        ],
        compiler_params=pltpu.CompilerParams(collective_id=0),
    )(x, Wdkv, Wuk, Wuv, Wkr, Wq, Wqr, Wo)
